# baseline (speedup 1.0000x reference)
import functools

import jax
import jax.numpy as jnp
from jax import lax
from jax.experimental import pallas as pl
from jax.experimental.pallas import tpu as pltpu
from jax.experimental.pallas import tpu_sc as plsc

_VOCAB = 272
_D = 512
_ADDR_KEY = 206
_CODE_START = 256
_CODE_END = 257

_NC = 2
_NS = 16
_NW = _NC * _NS
_L = 16


def _meta_tc(token_ids):
    B, S = token_ids.shape

    def body(tok_ref, meta_ref):
        tok = tok_ref[...]
        pos = lax.broadcasted_iota(jnp.int32, (B, S), 1)
        y = jnp.where(tok == _CODE_START, pos, -1)
        k = 1
        while k < S:
            shifted = jnp.concatenate(
                [jnp.full((B, k), -1, jnp.int32), y[:, : S - k]], axis=1
            )
            y = jnp.maximum(y, shifted)
            k *= 2
        first_ce = jnp.min(
            jnp.where(tok == _CODE_END, pos, S), axis=1, keepdims=True
        )
        mask = (y >= 0) & (pos < first_ce) & (tok < 256)
        addr = jnp.maximum(pos - y - 1, 0)
        meta = (
            (addr & 15)
            | (((addr >> 4) & 15) << 4)
            | (((addr >> 8) & 15) << 8)
            | jnp.where(mask, 1 << 12, 0)
        )
        meta_ref[...] = meta

    return pl.pallas_call(
        body, out_shape=jax.ShapeDtypeStruct((B, S), jnp.int32)
    )(token_ids)


def _sc_embed(tokens, meta, W):
    T = tokens.shape[0]
    per_w = T // _NW
    C = 64
    n_chunks = per_w // C
    NBUF = 3
    mesh = plsc.VectorSubcoreMesh(core_axis_name="c", subcore_axis_name="s")

    @functools.partial(
        pl.kernel,
        mesh=mesh,
        out_type=jax.ShapeDtypeStruct((T, _D), jnp.float32),
        compiler_params=pltpu.CompilerParams(needs_layout_passes=False),
        scratch_types=[
            pltpu.VMEM((per_w,), jnp.int32),
            pltpu.VMEM((per_w,), jnp.int32),
            [pltpu.VMEM((C, _D), jnp.float32) for _ in range(NBUF)],
            [pltpu.SemaphoreType.DMA for _ in range(NBUF)],
            [pltpu.SemaphoreType.DMA for _ in range(NBUF)],
        ],
    )
    def body(tok_hbm, meta_hbm, w_hbm, out_hbm, idx_v, meta_v, rows, gsem, ssem):
        wid = lax.axis_index("s") * _NC + lax.axis_index("c")
        base = wid * per_w
        pltpu.sync_copy(tok_hbm.at[pl.ds(base, per_w)], idx_v)
        pltpu.sync_copy(meta_hbm.at[pl.ds(base, per_w)], meta_v)
        ones = jnp.full((_L,), 1.0, jnp.float32)
        gdesc = [None] * NBUF
        sdesc = [None] * NBUF

        def fire_gather(i):
            b = i % NBUF
            gdesc[b] = pltpu.async_copy(
                w_hbm.at[idx_v.at[pl.ds(i * C, C)]], rows[b], gsem[b]
            )

        fire_gather(0)
        for i in range(n_chunks):
            b = i % NBUF
            if i + 1 < n_chunks:
                nb = (i + 1) % NBUF
                if sdesc[nb] is not None:
                    sdesc[nb].wait()
                fire_gather(i + 1)
            gdesc[b].wait()
            for g in range(C // _L):
                m = meta_v[pl.ds(i * C + g * _L, _L)]
                msk = (m >> 12) > 0
                ridx = lax.iota(jnp.int32, _L) + g * _L
                plsc.store_scatter(
                    rows[b], [ridx, _ADDR_KEY + (m & 15)], ones, mask=msk
                )
                plsc.store_scatter(
                    rows[b], [ridx, _ADDR_KEY + 16 + ((m >> 4) & 15)], ones, mask=msk
                )
                plsc.store_scatter(
                    rows[b], [ridx, _ADDR_KEY + 32 + ((m >> 8) & 15)], ones, mask=msk
                )
            sdesc[b] = pltpu.async_copy(
                rows[b], out_hbm.at[pl.ds(base + i * C, C)], ssem[b]
            )
        for b in range(NBUF):
            if sdesc[b] is not None:
                sdesc[b].wait()

    return body(tokens, meta, W)


_VPAD = 384


def _tc_embed(tok_col, meta_col, Wb):
    T2 = tok_col.shape[0]
    TB = 1024
    grid = (T2 // TB,)

    def body(tok_ref, meta_ref, w_ref, out_ref):
        tok = tok_ref[...]
        voc = lax.broadcasted_iota(jnp.int32, (TB, _VPAD), 1)
        onehot = (voc == tok).astype(jnp.bfloat16)
        x = jnp.dot(onehot, w_ref[...], preferred_element_type=jnp.float32)
        m = meta_ref[...]
        ci = lax.broadcasted_iota(jnp.int32, (TB, _D), 1)
        rel = ci - _ADDR_KEY
        grp = jnp.clip(lax.shift_right_arithmetic(rel, 4), 0, 2)
        sel = lax.shift_right_logical(m, grp * 4) & 15
        hit = (
            (rel >= 0)
            & (rel < 48)
            & ((m >> 12) > 0)
            & (sel == rel - (grp << 4))
        )
        out_ref[...] = jnp.where(hit, jnp.float32(1.0), x)

    return pl.pallas_call(
        body,
        grid=grid,
        in_specs=[
            pl.BlockSpec((TB, 1), lambda i: (i, 0)),
            pl.BlockSpec((TB, 1), lambda i: (i, 0)),
            pl.BlockSpec((_VPAD, _D), lambda i: (0, 0)),
        ],
        out_specs=pl.BlockSpec((TB, _D), lambda i: (i, 0)),
        out_shape=jax.ShapeDtypeStruct((T2, _D), jnp.float32),
    )(tok_col, meta_col, Wb)


_SC_TOKENS = 16384


def kernel(token_ids, W):
    B, S = token_ids.shape
    meta = _meta_tc(token_ids).reshape(-1)
    toks = token_ids.reshape(-1)
    K = _SC_TOKENS
    sc_out = _sc_embed(toks[:K], meta[:K], W)
    Wb = jnp.pad(W, ((0, _VPAD - _VOCAB), (0, 0))).astype(jnp.bfloat16)
    tc_out = _tc_embed(
        toks[K:].reshape(-1, 1), meta[K:].reshape(-1, 1), Wb
    )
    out = jnp.concatenate([sc_out, tc_out], axis=0)
    return out.reshape(B, S, _D)


# speedup vs baseline: 1.1140x; 1.1140x over previous
import functools

import jax
import jax.numpy as jnp
from jax import lax
from jax.experimental import pallas as pl
from jax.experimental.pallas import tpu as pltpu
from jax.experimental.pallas import tpu_sc as plsc

_VOCAB = 272
_D = 512
_ADDR_KEY = 206
_CODE_START = 256
_CODE_END = 257

_NC = 2
_NS = 16
_NW = _NC * _NS
_L = 16


def _meta_tc(token_ids):
    B, S = token_ids.shape

    def body(tok_ref, meta_ref):
        tok = tok_ref[...]
        pos = lax.broadcasted_iota(jnp.int32, (B, S), 1)
        y = jnp.where(tok == _CODE_START, pos, -1)
        k = 1
        while k < S:
            shifted = jnp.concatenate(
                [jnp.full((B, k), -1, jnp.int32), y[:, : S - k]], axis=1
            )
            y = jnp.maximum(y, shifted)
            k *= 2
        first_ce = jnp.min(
            jnp.where(tok == _CODE_END, pos, S), axis=1, keepdims=True
        )
        mask = (y >= 0) & (pos < first_ce) & (tok < 256)
        addr = jnp.maximum(pos - y - 1, 0)
        meta = (
            (addr & 15)
            | (((addr >> 4) & 15) << 4)
            | (((addr >> 8) & 15) << 8)
            | jnp.where(mask, 1 << 12, 0)
        )
        meta_ref[...] = meta

    return pl.pallas_call(
        body, out_shape=jax.ShapeDtypeStruct((B, S), jnp.int32)
    )(token_ids)


def _sc_embed(tokens, meta, W):
    T = tokens.shape[0]
    per_w = T // _NW
    C = 64
    n_chunks = per_w // C
    NBUF = 3
    mesh = plsc.VectorSubcoreMesh(core_axis_name="c", subcore_axis_name="s")

    @functools.partial(
        pl.kernel,
        mesh=mesh,
        out_type=jax.ShapeDtypeStruct((T, _D), jnp.float32),
        compiler_params=pltpu.CompilerParams(needs_layout_passes=False),
        scratch_types=[
            pltpu.VMEM((per_w,), jnp.int32),
            pltpu.VMEM((per_w,), jnp.int32),
            [pltpu.VMEM((C, _D), jnp.float32) for _ in range(NBUF)],
            [pltpu.SemaphoreType.DMA for _ in range(NBUF)],
            [pltpu.SemaphoreType.DMA for _ in range(NBUF)],
        ],
    )
    def body(tok_hbm, meta_hbm, w_hbm, out_hbm, idx_v, meta_v, rows, gsem, ssem):
        wid = lax.axis_index("s") * _NC + lax.axis_index("c")
        base = wid * per_w
        pltpu.sync_copy(tok_hbm.at[pl.ds(base, per_w)], idx_v)
        pltpu.sync_copy(meta_hbm.at[pl.ds(base, per_w)], meta_v)
        ones = jnp.full((_L,), 1.0, jnp.float32)
        gdesc = [None] * NBUF
        sdesc = [None] * NBUF

        def fire_gather(i):
            b = i % NBUF
            gdesc[b] = pltpu.async_copy(
                w_hbm.at[idx_v.at[pl.ds(i * C, C)]], rows[b], gsem[b]
            )

        fire_gather(0)
        for i in range(n_chunks):
            b = i % NBUF
            if i + 1 < n_chunks:
                nb = (i + 1) % NBUF
                if sdesc[nb] is not None:
                    sdesc[nb].wait()
                fire_gather(i + 1)
            gdesc[b].wait()
            for g in range(C // _L):
                m = meta_v[pl.ds(i * C + g * _L, _L)]
                msk = (m >> 12) > 0
                ridx = lax.iota(jnp.int32, _L) + g * _L
                plsc.store_scatter(
                    rows[b], [ridx, _ADDR_KEY + (m & 15)], ones, mask=msk
                )
                plsc.store_scatter(
                    rows[b], [ridx, _ADDR_KEY + 16 + ((m >> 4) & 15)], ones, mask=msk
                )
                plsc.store_scatter(
                    rows[b], [ridx, _ADDR_KEY + 32 + ((m >> 8) & 15)], ones, mask=msk
                )
            sdesc[b] = pltpu.async_copy(
                rows[b], out_hbm.at[pl.ds(base + i * C, C)], ssem[b]
            )
        for b in range(NBUF):
            if sdesc[b] is not None:
                sdesc[b].wait()

    return body(tokens, meta, W)


_VPAD = 384


def _tc_embed(tok_col, meta_col, Wb, K):
    T = tok_col.shape[0]
    TB = 1024
    off = K // TB
    grid = ((T - K) // TB,)

    def body(tok_ref, meta_ref, w_ref, out_ref):
        tok = tok_ref[...]
        voc = lax.broadcasted_iota(jnp.int32, (TB, _VPAD), 1)
        onehot = (voc == tok).astype(jnp.bfloat16)
        x = jnp.dot(onehot, w_ref[...], preferred_element_type=jnp.float32)
        m = meta_ref[...]
        ci = lax.broadcasted_iota(jnp.int32, (TB, _D), 1)
        rel = ci - _ADDR_KEY
        grp = jnp.clip(lax.shift_right_arithmetic(rel, 4), 0, 2)
        sel = lax.shift_right_logical(m, grp * 4) & 15
        hit = (
            (rel >= 0)
            & (rel < 48)
            & ((m >> 12) > 0)
            & (sel == rel - (grp << 4))
        )
        out_ref[...] = jnp.where(hit, jnp.float32(1.0), x)

    return pl.pallas_call(
        body,
        grid=grid,
        in_specs=[
            pl.BlockSpec((TB, 1), lambda i: (i + off, 0)),
            pl.BlockSpec((TB, 1), lambda i: (i + off, 0)),
            pl.BlockSpec((_VPAD, _D), lambda i: (0, 0)),
        ],
        out_specs=pl.BlockSpec((TB, _D), lambda i: (i + off, 0)),
        out_shape=jax.ShapeDtypeStruct((T, _D), jnp.float32),
    )(tok_col, meta_col, Wb)


_SC_TOKENS = 16384


def kernel(token_ids, W):
    B, S = token_ids.shape
    meta = _meta_tc(token_ids).reshape(-1)
    toks = token_ids.reshape(-1)
    K = _SC_TOKENS
    sc_out = _sc_embed(toks[:K], meta[:K], W)
    Wb = jnp.pad(W, ((0, _VPAD - _VOCAB), (0, 0))).astype(jnp.bfloat16)
    tc_full = _tc_embed(
        toks.reshape(-1, 1), meta.reshape(-1, 1), Wb, K
    )
    out = lax.dynamic_update_slice(tc_full, sc_out, (0, 0))
    return out.reshape(B, S, _D)


# speedup vs baseline: 1.1976x; 1.0751x over previous
import functools

import jax
import jax.numpy as jnp
from jax import lax
from jax.experimental import pallas as pl
from jax.experimental.pallas import tpu as pltpu
from jax.experimental.pallas import tpu_sc as plsc

_VOCAB = 272
_D = 512
_ADDR_KEY = 206
_CODE_START = 256
_CODE_END = 257

_NC = 2
_NS = 16
_NW = _NC * _NS
_L = 16


def _meta_tc(token_ids):
    B, S = token_ids.shape

    def body(tok_ref, meta_ref):
        tok = tok_ref[...]
        pos = lax.broadcasted_iota(jnp.int32, (B, S), 1)
        y = jnp.where(tok == _CODE_START, pos, -1)
        k = 1
        while k < S:
            shifted = jnp.concatenate(
                [jnp.full((B, k), -1, jnp.int32), y[:, : S - k]], axis=1
            )
            y = jnp.maximum(y, shifted)
            k *= 2
        first_ce = jnp.min(
            jnp.where(tok == _CODE_END, pos, S), axis=1, keepdims=True
        )
        mask = (y >= 0) & (pos < first_ce) & (tok < 256)
        addr = jnp.maximum(pos - y - 1, 0)
        meta = (
            (addr & 15)
            | (((addr >> 4) & 15) << 4)
            | (((addr >> 8) & 15) << 8)
            | jnp.where(mask, 1 << 12, 0)
        )
        meta_ref[...] = meta

    return pl.pallas_call(
        body, out_shape=jax.ShapeDtypeStruct((B, S), jnp.int32)
    )(token_ids)


def _sc_embed(tokens, meta, W):
    T = tokens.shape[0]
    per_w = T // _NW
    C = 64
    n_chunks = per_w // C
    NBUF = 3
    mesh = plsc.VectorSubcoreMesh(core_axis_name="c", subcore_axis_name="s")

    @functools.partial(
        pl.kernel,
        mesh=mesh,
        out_type=jax.ShapeDtypeStruct((T, _D), jnp.float32),
        compiler_params=pltpu.CompilerParams(needs_layout_passes=False),
        scratch_types=[
            pltpu.VMEM((per_w,), jnp.int32),
            pltpu.VMEM((per_w,), jnp.int32),
            [pltpu.VMEM((C, _D), jnp.float32) for _ in range(NBUF)],
            [pltpu.SemaphoreType.DMA for _ in range(NBUF)],
            [pltpu.SemaphoreType.DMA for _ in range(NBUF)],
        ],
    )
    def body(tok_hbm, meta_hbm, w_hbm, out_hbm, idx_v, meta_v, rows, gsem, ssem):
        wid = lax.axis_index("s") * _NC + lax.axis_index("c")
        base = wid * per_w
        pltpu.sync_copy(tok_hbm.at[pl.ds(base, per_w)], idx_v)
        pltpu.sync_copy(meta_hbm.at[pl.ds(base, per_w)], meta_v)
        ones = jnp.full((_L,), 1.0, jnp.float32)
        gdesc = [None] * NBUF
        sdesc = [None] * NBUF

        def fire_gather(i):
            b = i % NBUF
            gdesc[b] = pltpu.async_copy(
                w_hbm.at[idx_v.at[pl.ds(i * C, C)]], rows[b], gsem[b]
            )

        fire_gather(0)
        for i in range(n_chunks):
            b = i % NBUF
            if i + 1 < n_chunks:
                nb = (i + 1) % NBUF
                if sdesc[nb] is not None:
                    sdesc[nb].wait()
                fire_gather(i + 1)
            gdesc[b].wait()
            for g in range(C // _L):
                m = meta_v[pl.ds(i * C + g * _L, _L)]
                msk = (m >> 12) > 0
                ridx = lax.iota(jnp.int32, _L) + g * _L
                plsc.store_scatter(
                    rows[b], [ridx, _ADDR_KEY + (m & 15)], ones, mask=msk
                )
                plsc.store_scatter(
                    rows[b], [ridx, _ADDR_KEY + 16 + ((m >> 4) & 15)], ones, mask=msk
                )
                plsc.store_scatter(
                    rows[b], [ridx, _ADDR_KEY + 32 + ((m >> 8) & 15)], ones, mask=msk
                )
            sdesc[b] = pltpu.async_copy(
                rows[b], out_hbm.at[pl.ds(base + i * C, C)], ssem[b]
            )
        for b in range(NBUF):
            if sdesc[b] is not None:
                sdesc[b].wait()

    return body(tokens, meta, W)


def kernel(token_ids, W):
    B, S = token_ids.shape
    meta = _meta_tc(token_ids)
    out = _sc_embed(token_ids.reshape(-1), meta.reshape(-1), W)
    return out.reshape(B, S, _D)


# speedup vs baseline: 1.2050x; 1.0061x over previous
import functools

import jax
import jax.numpy as jnp
from jax import lax
from jax.experimental import pallas as pl
from jax.experimental.pallas import tpu as pltpu
from jax.experimental.pallas import tpu_sc as plsc

_VOCAB = 272
_D = 512
_ADDR_KEY = 206
_CODE_START = 256
_CODE_END = 257

_NC = 2
_NS = 16
_NW = _NC * _NS
_L = 16


def _meta_tc(token_ids):
    B, S = token_ids.shape

    def body(tok_ref, meta_ref):
        tok = tok_ref[...]
        pos = lax.broadcasted_iota(jnp.int32, (B, S), 1)
        y = jnp.where(tok == _CODE_START, pos, -1)
        k = 1
        while k < S:
            shifted = jnp.concatenate(
                [jnp.full((B, k), -1, jnp.int32), y[:, : S - k]], axis=1
            )
            y = jnp.maximum(y, shifted)
            k *= 2
        first_ce = jnp.min(
            jnp.where(tok == _CODE_END, pos, S), axis=1, keepdims=True
        )
        mask = (y >= 0) & (pos < first_ce) & (tok < 256)
        addr = jnp.maximum(pos - y - 1, 0)
        meta = (
            (addr & 15)
            | (((addr >> 4) & 15) << 4)
            | (((addr >> 8) & 15) << 8)
            | jnp.where(mask, 1 << 12, 0)
        )
        meta_ref[...] = meta

    return pl.pallas_call(
        body, out_shape=jax.ShapeDtypeStruct((B, S), jnp.int32)
    )(token_ids)


def _sc_embed(tokens, meta, W):
    T = tokens.shape[0]
    per_w = T // _NW
    C = 64
    n_chunks = per_w // C
    NBUF = 3
    mesh = plsc.VectorSubcoreMesh(core_axis_name="c", subcore_axis_name="s")

    @functools.partial(
        pl.kernel,
        mesh=mesh,
        out_type=jax.ShapeDtypeStruct((T, _D), jnp.float32),
        compiler_params=pltpu.CompilerParams(needs_layout_passes=False),
        scratch_types=[
            pltpu.VMEM((per_w,), jnp.int32),
            pltpu.VMEM((per_w,), jnp.int32),
            [pltpu.VMEM((C, _D), jnp.float32) for _ in range(NBUF)],
            [pltpu.SemaphoreType.DMA for _ in range(NBUF)],
            [pltpu.SemaphoreType.DMA for _ in range(NBUF)],
        ],
    )
    def body(tok_hbm, meta_hbm, w_hbm, out_hbm, idx_v, meta_v, rows, gsem, ssem):
        wid = lax.axis_index("s") * _NC + lax.axis_index("c")
        base = wid * per_w
        pltpu.sync_copy(tok_hbm.at[pl.ds(base, per_w)], idx_v)
        pltpu.sync_copy(meta_hbm.at[pl.ds(base, per_w)], meta_v)
        ones = jnp.full((_L,), 1.0, jnp.float32)
        gdesc = [None] * NBUF
        sdesc = [None] * NBUF

        def fire_gather(i):
            b = i % NBUF
            gdesc[b] = pltpu.async_copy(
                w_hbm.at[idx_v.at[pl.ds(i * C, C)]], rows[b], gsem[b]
            )

        fire_gather(0)
        fire_gather(1)
        for i in range(n_chunks):
            b = i % NBUF
            if i + 2 < n_chunks:
                nb = (i + 2) % NBUF
                if sdesc[nb] is not None:
                    sdesc[nb].wait()
                fire_gather(i + 2)
            gdesc[b].wait()
            for g in range(C // _L):
                m = meta_v[pl.ds(i * C + g * _L, _L)]
                msk = (m >> 12) > 0
                ridx = lax.iota(jnp.int32, _L) + g * _L
                plsc.store_scatter(
                    rows[b], [ridx, _ADDR_KEY + (m & 15)], ones, mask=msk
                )
                plsc.store_scatter(
                    rows[b], [ridx, _ADDR_KEY + 16 + ((m >> 4) & 15)], ones, mask=msk
                )
                plsc.store_scatter(
                    rows[b], [ridx, _ADDR_KEY + 32 + ((m >> 8) & 15)], ones, mask=msk
                )
            sdesc[b] = pltpu.async_copy(
                rows[b], out_hbm.at[pl.ds(base + i * C, C)], ssem[b]
            )
        for b in range(NBUF):
            if sdesc[b] is not None:
                sdesc[b].wait()

    return body(tokens, meta, W)


def kernel(token_ids, W):
    B, S = token_ids.shape
    meta = _meta_tc(token_ids)
    out = _sc_embed(token_ids.reshape(-1), meta.reshape(-1), W)
    return out.reshape(B, S, _D)


# speedup vs baseline: 1.2091x; 1.0034x over previous
import functools

import jax
import jax.numpy as jnp
from jax import lax
from jax.experimental import pallas as pl
from jax.experimental.pallas import tpu as pltpu
from jax.experimental.pallas import tpu_sc as plsc

_VOCAB = 272
_D = 512
_ADDR_KEY = 206
_CODE_START = 256
_CODE_END = 257

_NC = 2
_NS = 16
_NW = _NC * _NS
_L = 16


def _meta_tc(token_ids):
    B, S = token_ids.shape

    def body(tok_ref, meta_ref):
        tok = tok_ref[...]
        pos = lax.broadcasted_iota(jnp.int32, (B, S), 1)
        y = jnp.where(tok == _CODE_START, pos, -1)
        k = 1
        while k < S:
            shifted = jnp.concatenate(
                [jnp.full((B, k), -1, jnp.int32), y[:, : S - k]], axis=1
            )
            y = jnp.maximum(y, shifted)
            k *= 2
        first_ce = jnp.min(
            jnp.where(tok == _CODE_END, pos, S), axis=1, keepdims=True
        )
        mask = (y >= 0) & (pos < first_ce) & (tok < 256)
        addr = jnp.maximum(pos - y - 1, 0)
        meta = (
            (addr & 15)
            | (((addr >> 4) & 15) << 4)
            | (((addr >> 8) & 15) << 8)
            | jnp.where(mask, 1 << 12, 0)
        )
        meta_ref[...] = meta

    return pl.pallas_call(
        body, out_shape=jax.ShapeDtypeStruct((B, S), jnp.int32)
    )(token_ids)


def _sc_embed(tokens, meta, W):
    T = tokens.shape[0]
    per_w = T // _NW
    C = 80
    NBUF = 3
    offs, sizes = [], []
    o = 0
    while o < per_w:
        c = min(C, per_w - o)
        offs.append(o)
        sizes.append(c)
        o += c
    n_chunks = len(sizes)
    mesh = plsc.VectorSubcoreMesh(core_axis_name="c", subcore_axis_name="s")

    @functools.partial(
        pl.kernel,
        mesh=mesh,
        out_type=jax.ShapeDtypeStruct((T, _D), jnp.float32),
        compiler_params=pltpu.CompilerParams(needs_layout_passes=False),
        scratch_types=[
            pltpu.VMEM((per_w,), jnp.int32),
            pltpu.VMEM((per_w,), jnp.int32),
            [pltpu.VMEM((C, _D), jnp.float32) for _ in range(NBUF)],
            [pltpu.SemaphoreType.DMA for _ in range(NBUF)],
            [pltpu.SemaphoreType.DMA for _ in range(NBUF)],
        ],
    )
    def body(tok_hbm, meta_hbm, w_hbm, out_hbm, idx_v, meta_v, rows, gsem, ssem):
        wid = lax.axis_index("s") * _NC + lax.axis_index("c")
        base = wid * per_w
        pltpu.sync_copy(tok_hbm.at[pl.ds(base, per_w)], idx_v)
        pltpu.sync_copy(meta_hbm.at[pl.ds(base, per_w)], meta_v)
        ones = jnp.full((_L,), 1.0, jnp.float32)
        gdesc = [None] * NBUF
        sdesc = [None] * NBUF

        def buf_slice(b, sz):
            return rows[b] if sz == C else rows[b].at[pl.ds(0, sz)]

        def fire_gather(i):
            b = i % NBUF
            gdesc[b] = pltpu.async_copy(
                w_hbm.at[idx_v.at[pl.ds(offs[i], sizes[i])]],
                buf_slice(b, sizes[i]),
                gsem[b],
            )

        fire_gather(0)
        fire_gather(1)
        for i in range(n_chunks):
            b = i % NBUF
            if i + 2 < n_chunks:
                nb = (i + 2) % NBUF
                if sdesc[nb] is not None:
                    sdesc[nb].wait()
                fire_gather(i + 2)
            gdesc[b].wait()
            for g in range(sizes[i] // _L):
                m = meta_v[pl.ds(offs[i] + g * _L, _L)]
                msk = (m >> 12) > 0
                ridx = lax.iota(jnp.int32, _L) + g * _L
                plsc.store_scatter(
                    rows[b], [ridx, _ADDR_KEY + (m & 15)], ones, mask=msk
                )
                plsc.store_scatter(
                    rows[b], [ridx, _ADDR_KEY + 16 + ((m >> 4) & 15)], ones, mask=msk
                )
                plsc.store_scatter(
                    rows[b], [ridx, _ADDR_KEY + 32 + ((m >> 8) & 15)], ones, mask=msk
                )
            sdesc[b] = pltpu.async_copy(
                buf_slice(b, sizes[i]),
                out_hbm.at[pl.ds(base + offs[i], sizes[i])],
                ssem[b],
            )
        for b in range(NBUF):
            if sdesc[b] is not None:
                sdesc[b].wait()

    return body(tokens, meta, W)


def kernel(token_ids, W):
    B, S = token_ids.shape
    meta = _meta_tc(token_ids)
    out = _sc_embed(token_ids.reshape(-1), meta.reshape(-1), W)
    return out.reshape(B, S, _D)
